# baseline (speedup 1.0000x reference)
import jax
import jax.numpy as jnp
from jax.experimental import pallas as pl
from jax.experimental.pallas import tpu as pltpu


_BLK = 512


def _body(w_ref, o_hbm, sems):
    i = pl.program_id(0)
    batch = o_hbm.shape[0]
    for b in range(batch):
        pltpu.make_async_copy(
            w_ref, o_hbm.at[b, pl.ds(i * _BLK, _BLK), :], sems.at[b]
        ).start()
    for b in range(batch):
        pltpu.make_async_copy(
            w_ref, o_hbm.at[b, pl.ds(i * _BLK, _BLK), :], sems.at[b]
        ).wait()


def kernel(x, pe_weight):
    batch = x.shape[0]
    max_len, d_model = pe_weight.shape
    return pl.pallas_call(
        _body,
        grid=(max_len // _BLK,),
        in_specs=[pl.BlockSpec((_BLK, d_model), lambda i: (i, 0))],
        out_specs=pl.BlockSpec(memory_space=pl.ANY),
        out_shape=jax.ShapeDtypeStruct((batch, max_len, d_model), pe_weight.dtype),
        scratch_shapes=[pltpu.SemaphoreType.DMA((batch,))],
    )(pe_weight)


# speedup vs baseline: 1.2139x; 1.2139x over previous
import jax
import jax.numpy as jnp
from jax.experimental import pallas as pl
from jax.experimental.pallas import tpu as pltpu


_NCHUNK = 16


def _body(w_hbm, o_hbm, buf, in_sems, out_sem):
    n_rows, _ = w_hbm.shape
    batch = o_hbm.shape[0]
    chunk = n_rows // _NCHUNK

    def in_copy(c):
        sl = pl.ds(c * chunk, chunk)
        return pltpu.make_async_copy(w_hbm.at[sl, :], buf.at[sl, :], in_sems.at[c])

    def out_copy(c, b):
        sl = pl.ds(c * chunk, chunk)
        return pltpu.make_async_copy(buf.at[sl, :], o_hbm.at[b, sl, :], out_sem)

    for c in range(_NCHUNK):
        in_copy(c).start()
    for c in range(_NCHUNK):
        in_copy(c).wait()
        for b in range(batch):
            out_copy(c, b).start()
    for c in range(_NCHUNK):
        for b in range(batch):
            out_copy(c, b).wait()


def kernel(x, pe_weight):
    batch = x.shape[0]
    max_len, d_model = pe_weight.shape
    return pl.pallas_call(
        _body,
        in_specs=[pl.BlockSpec(memory_space=pl.ANY)],
        out_specs=pl.BlockSpec(memory_space=pl.ANY),
        out_shape=jax.ShapeDtypeStruct((batch, max_len, d_model), pe_weight.dtype),
        scratch_shapes=[
            pltpu.VMEM((max_len, d_model), pe_weight.dtype),
            pltpu.SemaphoreType.DMA((_NCHUNK,)),
            pltpu.SemaphoreType.DMA,
        ],
    )(pe_weight)
